# Initial kernel scaffold; baseline (speedup 1.0000x reference)
import jax
import jax.numpy as jnp
from jax.experimental import pallas as pl


def kernel(h_val, assign, cst_edges, LE, PE, num_val, num_cst, W1, b1, W2, ln1_g, ln1_b, W3, ln2_g, ln2_b):
    raise NotImplementedError("write your pallas kernel here")


# speedup vs baseline: 5.7488x; 5.7488x over previous
import functools

import jax
import jax.numpy as jnp
from jax import lax
from jax.experimental import pallas as pl
from jax.experimental.pallas import tpu as pltpu
from jax.experimental.pallas import tpu_sc as plsc

HID = 128
_NC, _NS = 2, 16
_NW = _NC * _NS
_LN_EPS = 1e-5


def _encode_body(h_ref, a_ref, w1t_ref, w1l_ref, b1_ref, w2t_ref, g1_ref,
                 bb1_ref, w3t_ref, g2_ref, bb2_ref, xval_ref, m_ref):
    h = h_ref[...]
    t = jnp.dot(h, w1t_ref[...], preferred_element_type=jnp.float32)
    t = t + a_ref[...] * w1l_ref[...] + b1_ref[...]
    t = jnp.maximum(t, 0.0)
    u = jnp.dot(t, w2t_ref[...], preferred_element_type=jnp.float32)
    mu = jnp.mean(u, axis=1, keepdims=True)
    var = jnp.mean((u - mu) ** 2, axis=1, keepdims=True)
    xv = (u - mu) * lax.rsqrt(var + _LN_EPS) * g1_ref[...] + bb1_ref[...]
    xval_ref[...] = xv
    y = jnp.dot(xv, w3t_ref[...], preferred_element_type=jnp.float32)
    mu2 = jnp.mean(y, axis=1, keepdims=True)
    var2 = jnp.mean((y - mu2) ** 2, axis=1, keepdims=True)
    m_ref[...] = (y - mu2) * lax.rsqrt(var2 + _LN_EPS) * g2_ref[...] + bb2_ref[...]


def _encode(h_val, assign, W1, b1, W2, ln1_g, ln1_b, W3, ln2_g, ln2_b):
    n = h_val.shape[0]
    blk = 1000
    assert n % blk == 0
    grid = (n // blk,)
    w1t = W1[:, :HID].T
    w1l = W1[:, HID].reshape(1, HID)
    row = lambda i: (i, 0)
    full = lambda i: (0, 0)
    x_val, m = pl.pallas_call(
        _encode_body,
        grid=grid,
        in_specs=[
            pl.BlockSpec((blk, HID), row),
            pl.BlockSpec((blk, 1), row),
            pl.BlockSpec((HID, HID), full),
            pl.BlockSpec((1, HID), full),
            pl.BlockSpec((1, HID), full),
            pl.BlockSpec((HID, HID), full),
            pl.BlockSpec((1, HID), full),
            pl.BlockSpec((1, HID), full),
            pl.BlockSpec((HID, 4 * HID), full),
            pl.BlockSpec((1, 4 * HID), full),
            pl.BlockSpec((1, 4 * HID), full),
        ],
        out_specs=[
            pl.BlockSpec((blk, HID), row),
            pl.BlockSpec((blk, 4 * HID), row),
        ],
        out_shape=[
            jax.ShapeDtypeStruct((n, HID), jnp.float32),
            jax.ShapeDtypeStruct((n, 4 * HID), jnp.float32),
        ],
    )(h_val, assign.reshape(n, 1), w1t, w1l, b1.reshape(1, HID),
      W2.T, ln1_g.reshape(1, HID), ln1_b.reshape(1, HID),
      W3.T, ln2_g.reshape(1, 4 * HID), ln2_b.reshape(1, 4 * HID))
    return x_val, m


@functools.lru_cache(maxsize=None)
def _make_sc(E, nseg):
    epw = E // _NW
    assert epw * _NW == E
    CH = 128
    full = epw // CH
    tail = epw - full * CH
    assert tail % 8 == 0
    seg_pw = (nseg // _NS) & ~7
    seg_rem = nseg - seg_pw * _NS
    assert seg_rem % 8 == 0
    nz_full = seg_pw // CH
    nz_rem = seg_pw - nz_full * CH

    mesh = plsc.VectorSubcoreMesh(core_axis_name="c", subcore_axis_name="s",
                                  num_cores=_NC, num_subcores=_NS)

    scratch = [
        pltpu.VMEM((CH,), jnp.int32),
        pltpu.VMEM((CH,), jnp.int32),
        pltpu.VMEM((CH, HID), jnp.float32),
        pltpu.VMEM_SHARED((nseg, HID), jnp.float32),
        pltpu.SemaphoreType.DMA,
    ]
    if tail:
        scratch += [
            pltpu.VMEM((tail,), jnp.int32),
            pltpu.VMEM((tail,), jnp.int32),
            pltpu.VMEM((tail, HID), jnp.float32),
        ]

    @functools.partial(
        pl.kernel,
        mesh=mesh,
        out_type=jax.ShapeDtypeStruct((_NC * nseg, HID), jnp.float32),
        scratch_types=scratch,
    )
    def sc_kernel(mval_hbm, oidx_hbm, iidx_hbm, out_hbm,
                  oid_v, iid_v, rows_v, acc, sem, *tailrefs):
        cid = lax.axis_index("c")
        sid = lax.axis_index("s")
        wid = sid * _NC + cid

        def zrow(r, _):
            for k in range(HID // 16):
                rows_v[r, pl.ds(k * 16, 16)] = jnp.zeros((16,), jnp.float32)
            return 0
        lax.fori_loop(0, CH, zrow, 0)
        base_seg = pl.multiple_of(sid * seg_pw, 8)
        for t in range(nz_full):
            pltpu.sync_copy(rows_v, acc.at[pl.ds(base_seg + t * CH, CH)])
        if nz_rem:
            pltpu.sync_copy(rows_v.at[pl.ds(0, nz_rem)],
                            acc.at[pl.ds(base_seg + nz_full * CH, nz_rem)])
        if seg_rem:
            @pl.when(sid == 0)
            def _():
                pltpu.sync_copy(rows_v.at[pl.ds(0, seg_rem)],
                                acc.at[pl.ds(_NS * seg_pw, seg_rem)])
        plsc.subcore_barrier()

        base_e = pl.multiple_of(wid * epw, 8)

        def body(j, _):
            off = pl.multiple_of(base_e + j * CH, 8)
            pltpu.sync_copy(oidx_hbm.at[pl.ds(off, CH)], oid_v)
            pltpu.sync_copy(iidx_hbm.at[pl.ds(off, CH)], iid_v)
            pltpu.async_copy(mval_hbm.at[oid_v], rows_v, sem).wait()
            pltpu.sync_copy(rows_v, acc.at[iid_v], add=True)
            return 0
        lax.fori_loop(0, full, body, 0)

        if tail:
            oidt_v, iidt_v, rowst_v = tailrefs
            off = pl.multiple_of(base_e + full * CH, 8)
            pltpu.sync_copy(oidx_hbm.at[pl.ds(off, tail)], oidt_v)
            pltpu.sync_copy(iidx_hbm.at[pl.ds(off, tail)], iidt_v)
            pltpu.async_copy(mval_hbm.at[oidt_v], rowst_v, sem).wait()
            pltpu.sync_copy(rowst_v, acc.at[iidt_v], add=True)

        plsc.subcore_barrier()
        out_base = pl.multiple_of(cid * nseg + base_seg, 8)
        pltpu.sync_copy(acc.at[pl.ds(base_seg, seg_pw)],
                        out_hbm.at[pl.ds(out_base, seg_pw)])
        if seg_rem:
            @pl.when(sid == 0)
            def _():
                rem_base = pl.multiple_of(cid * nseg + _NS * seg_pw, 8)
                pltpu.sync_copy(acc.at[pl.ds(_NS * seg_pw, seg_rem)],
                                out_hbm.at[pl.ds(rem_base, seg_rem)])

    return sc_kernel


def _add_body(p_ref, o_ref):
    o_ref[...] = p_ref[0] + p_ref[1]


def _add_partials(partials, n):
    blk = 1000
    return pl.pallas_call(
        _add_body,
        grid=(n // blk,),
        in_specs=[pl.BlockSpec((2, blk, HID), lambda i: (0, i, 0))],
        out_specs=pl.BlockSpec((blk, HID), lambda i: (i, 0)),
        out_shape=jax.ShapeDtypeStruct((n, HID), jnp.float32),
    )(partials)


def kernel(h_val, assign, cst_edges, LE, PE, num_val, num_cst,
           W1, b1, W2, ln1_g, ln1_b, W3, ln2_g, ln2_b):
    n = h_val.shape[0]
    E = cst_edges.shape[1]
    x_val, m = _encode(h_val, assign, W1, b1, W2, ln1_g, ln1_b,
                       W3, ln2_g, ln2_b)
    m_val = m.reshape(4 * n, HID)
    oidx = (cst_edges[1].astype(jnp.int32) * 4
            + LE.astype(jnp.int32) * 2 + PE.astype(jnp.int32))
    iidx = cst_edges[0].astype(jnp.int32)
    partials = _make_sc(E, n)(m_val, oidx, iidx)
    r_cst = _add_partials(partials.reshape(2, n, HID), n)
    return (r_cst, x_val)
